# Initial kernel scaffold; baseline (speedup 1.0000x reference)
import jax
import jax.numpy as jnp
from jax.experimental import pallas as pl


def kernel(x, node_type, sensor_id, edge_index, edge_attr, type_emb, sensor_emb, enc_W1, enc_b1, enc_W2, enc_b2, g1_Wl, g1_bl, g1_Wr, g1_br, g1_We, g1_att, g1_bias, g2_Wl, g2_bl, g2_Wr, g2_br, g2_We, g2_att, g2_bias, gru_Wx, gru_bx, gru_Wh, gru_bh, ln_g, ln_b, dec_W1, dec_b1, dec_W2, dec_b2):
    raise NotImplementedError("write your pallas kernel here")


# speedup vs baseline: 22.9587x; 22.9587x over previous
import functools

import numpy as np
import jax
import jax.numpy as jnp
from jax import lax
from jax.experimental import pallas as pl
from jax.experimental.pallas import tpu as pltpu
from jax.experimental.pallas import tpu_sc as plsc

N = 50000
E = 800000
HID = 64
H = 4
C = 16
EDIM = 7

NW = 32
SUB = 128
GROUP = 1024
EPAD = 819200
NHALF = N // 2
ZROWS = 1568
ACC_ROWS = 16 * ZROWS
LAST_ROWS = NHALF - 15 * ZROWS

_f32 = jnp.float32

_G = np.kron(np.eye(H), np.ones((C, 1))).astype(np.float32)
_HB = np.kron(np.eye(H), np.ones((1, C))).astype(np.float32)
_P = np.eye(H, 16).astype(np.float32)
_Q = np.kron(np.eye(4, dtype=np.float32), np.ones((1, C), np.float32))
_Q = np.concatenate([_Q, np.zeros((12, 64), np.float32)], 0)

def _make_gather(width):
  nper = EPAD // NW
  ngroups = nper // GROUP

  @functools.partial(
      pl.kernel,
      mesh=plsc.VectorSubcoreMesh(core_axis_name="c", subcore_axis_name="s"),
      compiler_params=pltpu.CompilerParams(use_tc_tiling_on_sc=False),
      out_type=jax.ShapeDtypeStruct((EPAD, width), _f32),
      scratch_types=[
          pltpu.VMEM((GROUP,), jnp.int32),
          pltpu.VMEM((GROUP, width), _f32),
          pltpu.SemaphoreType.DMA,
      ],
  )
  def k(table, idx, out, idxv, rows, sem):
    wid = lax.axis_index("c") * 16 + lax.axis_index("s")
    base_w = wid * nper

    def body(g, carry):
      base = base_w + g * GROUP
      pltpu.sync_copy(idx.at[pl.ds(base, GROUP)], idxv)
      cps = [
          pltpu.async_copy(
              table.at[idxv.at[pl.ds(b * SUB, SUB)]],
              rows.at[pl.ds(b * SUB, SUB)],
              sem,
          )
          for b in range(GROUP // SUB)
      ]
      for cp in cps:
        cp.wait()
      pltpu.sync_copy(rows, out.at[pl.ds(base, GROUP)])
      return carry

    lax.fori_loop(0, ngroups, body, 0)

  return k


@functools.cache
def _gather64():
  return _make_gather(64)


@functools.cache
def _gather16():
  return _make_gather(16)


def _make_scatter(width, group):
  nper = EPAD // 16
  ngroups = nper // group

  @functools.partial(
      pl.kernel,
      mesh=plsc.VectorSubcoreMesh(core_axis_name="c", subcore_axis_name="s"),
      compiler_params=pltpu.CompilerParams(use_tc_tiling_on_sc=False),
      out_type=jax.ShapeDtypeStruct((N, width), _f32),
      scratch_types=[
          pltpu.VMEM_SHARED((ACC_ROWS, width), _f32),
          pltpu.VMEM((group, width), _f32),
          pltpu.VMEM((group // SUB, SUB), jnp.int32),
          pltpu.SemaphoreType.DMA,
      ],
  )
  def k(w_hbm, dst2d, zrows, acc_o, acc, wbuf, idx2, sem):
    c = lax.axis_index("c")
    s = lax.axis_index("s")
    pltpu.sync_copy(zrows, acc.at[pl.ds(s * ZROWS, ZROWS)])
    plsc.subcore_barrier()

    nb = c * NHALF

    def body(g, carry):
      base = s * nper + g * group
      row0 = base // SUB
      pltpu.sync_copy(dst2d.at[pl.ds(row0, group // SUB)], idx2)
      for r in range(group // SUB):
        for q in range(SUB // 16):
          v = idx2[r, pl.ds(q * 16, 16)]
          local = v - nb
          ok = (local >= 0) & (local < NHALF)
          dump = NHALF + (lax.iota(jnp.int32, 16) & 7)
          idx2[r, pl.ds(q * 16, 16)] = jnp.where(ok, local, dump)
      pltpu.sync_copy(w_hbm.at[pl.ds(base, group)], wbuf)
      for b in range(group // SUB):
        pltpu.sync_copy(wbuf.at[pl.ds(b * SUB, SUB)], acc.at[idx2.at[b]],
                        add=True)
      return carry

    lax.fori_loop(0, ngroups, body, 0)
    plsc.subcore_barrier()

    hb = c * NHALF

    @pl.when(s < 15)
    def _():
      pltpu.sync_copy(acc.at[pl.ds(s * ZROWS, ZROWS)],
                      acc_o.at[pl.ds(hb + s * ZROWS, ZROWS)])

    @pl.when(s == 15)
    def _():
      pltpu.sync_copy(acc.at[pl.ds(15 * ZROWS, LAST_ROWS)],
                      acc_o.at[pl.ds(hb + 15 * ZROWS, LAST_ROWS)])

  return k


@functools.cache
def _scatter64():
  return _make_scatter(64, 256)


@functools.cache
def _scatter16():
  return _make_scatter(16, 2048)


BN = 2000
BE = 8192
BE5 = 8000


def _full(shape):
  return pl.BlockSpec(shape, lambda i: tuple(0 for _ in shape))


def _rows(block, width):
  return pl.BlockSpec((block, width), lambda i: (i, 0))


def _t1_body(x, nt, sid, temb, semb, w1, b1, w2, b2, wl, bl, wr, br,
             h_o, xl_o, xr_o):
  xx = x[...]
  oh_t = (nt[...] == lax.broadcasted_iota(jnp.int32, (BN, 2), 1)).astype(_f32)
  oh_s = (sid[...] == lax.broadcasted_iota(jnp.int32, (BN, 6), 1)).astype(_f32)
  hcat = jnp.concatenate([xx, oh_t @ temb[...], oh_s @ semb[...]], axis=1)
  h = jnp.maximum(hcat @ w1[...] + b1[...], 0.0) @ w2[...] + b2[...]
  h_o[...] = h
  xl_o[...] = h @ wl[...] + bl[...]
  xr_o[...] = h @ wr[...] + br[...]


def _t1(x, nt, sid, temb, semb, w1, b1, w2, b2, wl, bl, wr, br):
  return pl.pallas_call(
      _t1_body,
      grid=(N // BN,),
      in_specs=[
          _rows(BN, 7), _rows(BN, 1), _rows(BN, 1),
          _full((2, 8)), _full((6, 8)),
          _full((23, 64)), _full((1, 64)), _full((64, 64)), _full((1, 64)),
          _full((64, 64)), _full((1, 64)), _full((64, 64)), _full((1, 64)),
      ],
      out_specs=[_rows(BN, 64), _rows(BN, 64), _rows(BN, 64)],
      out_shape=[jax.ShapeDtypeStruct((N, 64), _f32)] * 3,
  )(x, nt, sid, temb, semb, w1, b1, w2, b2, wl, bl, wr, br)


def _t2_body(xls, xrd, ea, we, attf, g, hbm, p, w_o, e_o):
  xl = xls[...]
  proj = ea[...] @ we[...]
  t = xl + xrd[...] + proj
  m = jnp.where(t > 0.0, t, 0.2 * t)
  ex = jnp.exp((m * attf[...]) @ g[...])
  w_o[...] = xl * (ex @ hbm[...])
  e_o[...] = ex @ p[...]


def _t2(xls, xrd, ea, we, attf):
  return pl.pallas_call(
      _t2_body,
      grid=(EPAD // BE,),
      in_specs=[
          _rows(BE, 64), _rows(BE, 64), _rows(BE, 7),
          _full((7, 64)), _full((1, 64)),
          _full((64, 4)), _full((4, 64)), _full((4, 16)),
      ],
      out_specs=[_rows(BE, 64), _rows(BE, 16)],
      out_shape=[
          jax.ShapeDtypeStruct((EPAD, 64), _f32),
          jax.ShapeDtypeStruct((EPAD, 16), _f32),
      ],
  )(xls, xrd, ea, we, attf, jnp.asarray(_G), jnp.asarray(_HB),
    jnp.asarray(_P))


def _t3_body(accw, accex, q, bias, wl, bl, wr, br, xl_o, xr_o):
  den = accex[...] @ q[...] + 1e-16
  g = accw[...] / den + bias[...]
  h2 = jnp.maximum(g, 0.0)
  xl_o[...] = h2 @ wl[...] + bl[...]
  xr_o[...] = h2 @ wr[...] + br[...]


def _t3(accw, accex, bias, wl, bl, wr, br):
  return pl.pallas_call(
      _t3_body,
      grid=(N // BN,),
      in_specs=[
          _rows(BN, 64), _rows(BN, 16), _full((16, 64)), _full((1, 64)),
          _full((64, 64)), _full((1, 64)), _full((64, 64)), _full((1, 64)),
      ],
      out_specs=[_rows(BN, 64), _rows(BN, 64)],
      out_shape=[jax.ShapeDtypeStruct((N, 64), _f32)] * 2,
  )(accw, accex, jnp.asarray(_Q), bias, wl, bl, wr, br)


def _t4_body(accw, accex, q, bias, wxr, wxz, wxn, brr, brz, bxn, bhn,
             lng, lnb, dw1, db1, dw2, db2, out_o, nh_o):
  den = accex[...] @ q[...] + 1e-16
  hg = accw[...] / den + bias[...]
  r = jax.nn.sigmoid(hg @ wxr[...] + brr[...])
  z = jax.nn.sigmoid(hg @ wxz[...] + brz[...])
  n = jnp.tanh(hg @ wxn[...] + bxn[...] + r * bhn[...])
  nh = (1.0 - z) * n
  mu = jnp.mean(nh, axis=1, keepdims=True)
  var = jnp.mean((nh - mu) ** 2, axis=1, keepdims=True)
  nh = (nh - mu) / jnp.sqrt(var + 1e-5) * lng[...] + lnb[...]
  nh_o[...] = nh
  out_o[...] = jnp.maximum(nh @ dw1[...] + db1[...], 0.0) @ dw2[...] + db2[...]


def _t4(accw, accex, bias, wxr, wxz, wxn, brr, brz, bxn, bhn,
        lng, lnb, dw1, db1, dw2, db2):
  return pl.pallas_call(
      _t4_body,
      grid=(N // BN,),
      in_specs=[
          _rows(BN, 64), _rows(BN, 16), _full((16, 64)), _full((1, 64)),
          _full((64, 64)), _full((64, 64)), _full((64, 64)),
          _full((1, 64)), _full((1, 64)), _full((1, 64)), _full((1, 64)),
          _full((1, 64)), _full((1, 64)),
          _full((64, 64)), _full((1, 64)), _full((64, 7)), _full((1, 7)),
      ],
      out_specs=[_rows(BN, 7), _rows(BN, 64)],
      out_shape=[
          jax.ShapeDtypeStruct((N, 7), _f32),
          jax.ShapeDtypeStruct((N, 64), _f32),
      ],
  )(accw, accex, jnp.asarray(_Q), bias, wxr, wxz, wxn, brr, brz, bxn, bhn,
    lng, lnb, dw1, db1, dw2, db2)


def _t5_body(ex, dend, a_o):
  a_o[...] = ex[:, 0:4] / (dend[:, 0:4] + 1e-16)


def _t5(exmat, dend):
  return pl.pallas_call(
      _t5_body,
      grid=(E // BE5,),
      in_specs=[_rows(BE5, 16), _rows(BE5, 16)],
      out_specs=_rows(BE5, 4),
      out_shape=jax.ShapeDtypeStruct((E, 4), _f32),
  )(exmat, dend)


def kernel(x, node_type, sensor_id, edge_index, edge_attr, type_emb,
           sensor_emb, enc_W1, enc_b1, enc_W2, enc_b2,
           g1_Wl, g1_bl, g1_Wr, g1_br, g1_We, g1_att, g1_bias,
           g2_Wl, g2_bl, g2_Wr, g2_br, g2_We, g2_att, g2_bias,
           gru_Wx, gru_bx, gru_Wh, gru_bh, ln_g, ln_b,
           dec_W1, dec_b1, dec_W2, dec_b2):
  src = edge_index[0].astype(jnp.int32)
  dst = edge_index[1].astype(jnp.int32)
  pad = EPAD - E
  zi = jnp.zeros((pad,), jnp.int32)
  src_g = jnp.concatenate([src, zi])
  dst_g = jnp.concatenate([dst, zi])
  dst_s = jnp.concatenate([dst, jnp.full((pad,), N, jnp.int32)])
  dst2d = dst_s.reshape(EPAD // SUB, SUB)
  ea_pad = jnp.concatenate([edge_attr, jnp.zeros((pad, EDIM), _f32)], axis=0)
  z64 = jnp.zeros((ZROWS, 64), _f32)
  z16 = jnp.zeros((ZROWS, 16), _f32)

  nt2 = node_type.astype(jnp.int32).reshape(N, 1)
  sid2 = sensor_id.astype(jnp.int32).reshape(N, 1)

  def row(v):
    return v.reshape(1, -1)

  h, xl1, xr1 = _t1(x, nt2, sid2, type_emb, sensor_emb,
                    enc_W1, row(enc_b1), enc_W2, row(enc_b2),
                    g1_Wl, row(g1_bl), g1_Wr, row(g1_br))

  xls1 = _gather64()(xl1, src_g)
  xrd1 = _gather64()(xr1, dst_g)
  w1, e1 = _t2(xls1, xrd1, ea_pad, g1_We, row(g1_att))
  accw1 = _scatter64()(w1, dst2d, z64)
  accex1 = _scatter16()(e1, dst2d, z16)

  xl2, xr2 = _t3(accw1, accex1, row(g1_bias),
                 g2_Wl, row(g2_bl), g2_Wr, row(g2_br))

  xls2 = _gather64()(xl2, src_g)
  xrd2 = _gather64()(xr2, dst_g)
  w2, e2 = _t2(xls2, xrd2, ea_pad, g2_We, row(g2_att))
  accw2 = _scatter64()(w2, dst2d, z64)
  accex2 = _scatter16()(e2, dst2d, z16)

  dend = _gather16()(accex2, dst_g)
  alpha2 = _t5(e2, dend[:E])

  out, new_hidden = _t4(
      accw2, accex2, row(g2_bias),
      gru_Wx[:, 0:64], gru_Wx[:, 64:128], gru_Wx[:, 128:192],
      row(gru_bx[0:64] + gru_bh[0:64]),
      row(gru_bx[64:128] + gru_bh[64:128]),
      row(gru_bx[128:192]), row(gru_bh[128:192]),
      row(ln_g), row(ln_b), dec_W1, row(dec_b1), dec_W2, row(dec_b2))

  return out, new_hidden, alpha2
